# baseline (speedup 1.0000x reference)
import functools

import jax
import jax.numpy as jnp
from jax.experimental import pallas as pl
from jax.experimental.pallas import tpu as pltpu

K = 50
BR = 256
CH = 256
KPAD = 64
NEG = -1e38


def _rough_topk_kernel(rows_ref, w_ref, b_ref, ments_ref,
                       vals_ref, idx_ref, s_ref):
    n = ments_ref.shape[0]
    r0 = pl.program_id(0) * BR
    nch = (r0 + BR + CH - 1) // CH
    nch4 = (nch + 3) // 4

    t = jax.lax.dot_general(
        rows_ref[...], w_ref[...], (((1,), (1,)), ((), ())),
        preferred_element_type=jnp.float32)
    t = t + b_ref[...]

    colf_ch = jax.lax.broadcasted_iota(
        jnp.int32, (CH, BR), 0).astype(jnp.float32)
    rowf = (jnp.float32(r0)
            + jax.lax.broadcasted_iota(
                jnp.int32, (CH, BR), 1).astype(jnp.float32))

    def fill(c, _):
        c0 = c * CH
        mc = ments_ref[pl.ds(c0, CH), :]
        s = jax.lax.dot_general(
            mc, t, (((1,), (1,)), ((), ())),
            preferred_element_type=jnp.float32)
        colf = jnp.float32(c0) + colf_ch
        s_ref[pl.ds(c0, CH), :] = jnp.where(colf < rowf, s, NEG)
        return 0

    jax.lax.fori_loop(0, 4 * nch4, fill, 0)

    nf = jnp.float32(n)

    def extract2(k, prev):
        ip1, ip2 = prev

        def subchunk(c0, carry):
            M1, I1, M2, I2 = carry
            chunk = s_ref[pl.ds(c0, CH), :]
            colf = jnp.float32(c0) + colf_ch
            hitp = (colf == ip1) | (colf == ip2)
            masked = jnp.where(hitp, -jnp.inf, chunk)
            s_ref[pl.ds(c0, CH), :] = masked
            m1 = jnp.max(masked, axis=0, keepdims=True)
            i1 = jnp.min(jnp.where(masked == m1, colf, nf),
                         axis=0, keepdims=True)
            masked2 = jnp.where(colf == i1, -jnp.inf, masked)
            m2 = jnp.max(masked2, axis=0, keepdims=True)
            i2 = jnp.min(jnp.where(masked2 == m2, colf, nf),
                         axis=0, keepdims=True)
            g = (M1 > m1) | ((M1 == m1) & (I1 < i1))
            n1v = jnp.where(g, M1, m1)
            n1i = jnp.where(g, I1, i1)
            lv = jnp.where(g, m1, M1)
            li = jnp.where(g, i1, I1)
            wv = jnp.where(g, M2, m2)
            wi = jnp.where(g, I2, i2)
            h = (lv > wv) | ((lv == wv) & (li < wi))
            return (n1v, n1i,
                    jnp.where(h, lv, wv), jnp.where(h, li, wi))

        def scan(c, carry):
            carry = subchunk((4 * c) * CH, carry)
            carry = subchunk((4 * c + 1) * CH, carry)
            carry = subchunk((4 * c + 2) * CH, carry)
            carry = subchunk((4 * c + 3) * CH, carry)
            return carry

        neginf = jnp.full((1, BR), -jnp.inf, jnp.float32)
        nfull = jnp.full((1, BR), nf, jnp.float32)
        v1, x1, v2, x2 = jax.lax.fori_loop(
            0, nch4, scan, (neginf, nfull, neginf, nfull))

        vals_ref[pl.ds(2 * k, 1), :] = jnp.where(v1 <= NEG * 0.5,
                                                 -jnp.inf, v1)
        idx_ref[pl.ds(2 * k, 1), :] = x1.astype(jnp.int32)
        vals_ref[pl.ds(2 * k + 1, 1), :] = jnp.where(v2 <= NEG * 0.5,
                                                     -jnp.inf, v2)
        idx_ref[pl.ds(2 * k + 1, 1), :] = x2.astype(jnp.int32)
        return (x1, x2)

    jax.lax.fori_loop(0, K // 2, extract2,
                      (jnp.full((1, BR), -1.0, jnp.float32),
                       jnp.full((1, BR), -2.0, jnp.float32)))
    vals_ref[pl.ds(K, KPAD - K), :] = jnp.zeros((KPAD - K, BR), jnp.float32)
    idx_ref[pl.ds(K, KPAD - K), :] = jnp.zeros((KPAD - K, BR), jnp.int32)


@jax.jit
def kernel(mentions, W, b):
    n, d = mentions.shape
    grid = (n // BR,)
    vals_t, idx_t = pl.pallas_call(
        _rough_topk_kernel,
        grid=grid,
        in_specs=[
            pl.BlockSpec((BR, d), lambda i: (i, 0)),
            pl.BlockSpec((d, d), lambda i: (0, 0)),
            pl.BlockSpec((1, d), lambda i: (0, 0)),
            pl.BlockSpec((n, d), lambda i: (0, 0)),
        ],
        out_specs=[
            pl.BlockSpec((KPAD, BR), lambda i: (0, i)),
            pl.BlockSpec((KPAD, BR), lambda i: (0, i)),
        ],
        out_shape=[
            jax.ShapeDtypeStruct((KPAD, n), jnp.float32),
            jax.ShapeDtypeStruct((KPAD, n), jnp.int32),
        ],
        scratch_shapes=[
            pltpu.VMEM((n, BR), jnp.float32),
        ],
    )(mentions, W, b.reshape(1, d), mentions)
    return vals_t[:K, :].T, idx_t[:K, :].T


# speedup vs baseline: 1.0104x; 1.0104x over previous
import functools

import jax
import jax.numpy as jnp
from jax.experimental import pallas as pl
from jax.experimental.pallas import tpu as pltpu

K = 50
BR = 256
CH = 128
UNROLL = 8
KPAD = 64
NEG = -1e38


def _rough_topk_kernel(rows_ref, w_ref, b_ref, ments_ref,
                       vals_ref, idx_ref, s_ref):
    n = ments_ref.shape[0]
    r0 = pl.program_id(0) * BR
    nch = (r0 + BR + CH - 1) // CH
    nchu = (nch + UNROLL - 1) // UNROLL

    t = jax.lax.dot_general(
        rows_ref[...], w_ref[...], (((1,), (1,)), ((), ())),
        preferred_element_type=jnp.float32)
    t = t + b_ref[...]

    colf_ch = jax.lax.broadcasted_iota(
        jnp.int32, (CH, BR), 0).astype(jnp.float32)
    rowf = (jnp.float32(r0)
            + jax.lax.broadcasted_iota(
                jnp.int32, (CH, BR), 1).astype(jnp.float32))

    def fill(c, _):
        c0 = c * CH
        mc = ments_ref[pl.ds(c0, CH), :]
        s = jax.lax.dot_general(
            mc, t, (((1,), (1,)), ((), ())),
            preferred_element_type=jnp.float32)
        colf = jnp.float32(c0) + colf_ch
        s_ref[pl.ds(c0, CH), :] = jnp.where(colf < rowf, s, NEG)
        return 0

    jax.lax.fori_loop(0, UNROLL * nchu, fill, 0)

    def extract(k, idx_prevf):
        def subchunk(c0, carry):
            m_run, i_run = carry
            chunk = s_ref[pl.ds(c0, CH), :]
            colf = jnp.float32(c0) + colf_ch
            masked = jnp.where(colf == idx_prevf, -jnp.inf, chunk)
            s_ref[pl.ds(c0, CH), :] = masked
            cmax = jnp.max(masked, axis=0, keepdims=True)
            cidx = jnp.min(jnp.where(masked == cmax, colf, jnp.float32(n)),
                           axis=0, keepdims=True)
            upd = cmax > m_run
            return (jnp.where(upd, cmax, m_run),
                    jnp.where(upd, cidx, i_run))

        def scan(c, carry):
            for h in range(UNROLL):
                carry = subchunk((UNROLL * c + h) * CH, carry)
            return carry

        m, idxf = jax.lax.fori_loop(
            0, nchu, scan,
            (jnp.full((1, BR), -jnp.inf, jnp.float32),
             jnp.full((1, BR), jnp.float32(n), jnp.float32)))

        vals_ref[pl.ds(k, 1), :] = jnp.where(m <= NEG * 0.5, -jnp.inf, m)
        idx_ref[pl.ds(k, 1), :] = idxf.astype(jnp.int32)
        return idxf

    jax.lax.fori_loop(0, K, extract, jnp.full((1, BR), -1.0, jnp.float32))
    vals_ref[pl.ds(K, KPAD - K), :] = jnp.zeros((KPAD - K, BR), jnp.float32)
    idx_ref[pl.ds(K, KPAD - K), :] = jnp.zeros((KPAD - K, BR), jnp.int32)


@jax.jit
def kernel(mentions, W, b):
    n, d = mentions.shape
    grid = (n // BR,)
    vals_t, idx_t = pl.pallas_call(
        _rough_topk_kernel,
        grid=grid,
        in_specs=[
            pl.BlockSpec((BR, d), lambda i: (i, 0)),
            pl.BlockSpec((d, d), lambda i: (0, 0)),
            pl.BlockSpec((1, d), lambda i: (0, 0)),
            pl.BlockSpec((n, d), lambda i: (0, 0)),
        ],
        out_specs=[
            pl.BlockSpec((KPAD, BR), lambda i: (0, i)),
            pl.BlockSpec((KPAD, BR), lambda i: (0, i)),
        ],
        out_shape=[
            jax.ShapeDtypeStruct((KPAD, n), jnp.float32),
            jax.ShapeDtypeStruct((KPAD, n), jnp.int32),
        ],
        scratch_shapes=[
            pltpu.VMEM((n, BR), jnp.float32),
        ],
    )(mentions, W, b.reshape(1, d), mentions)
    return vals_t[:K, :].T, idx_t[:K, :].T


# speedup vs baseline: 1.0756x; 1.0645x over previous
import functools

import jax
import jax.numpy as jnp
from jax.experimental import pallas as pl
from jax.experimental.pallas import tpu as pltpu

K = 50
BR = 256
CH = 256
UNROLL = 4
KPAD = 64
NEG = -1e38


def _rough_topk_kernel(rows_ref, w_ref, b_ref, ments_ref,
                       vals_ref, idx_ref, s_ref):
    n = ments_ref.shape[0]
    r0 = pl.program_id(0) * BR
    nch = (r0 + BR + CH - 1) // CH
    nchu = (nch + UNROLL - 1) // UNROLL

    t = jax.lax.dot_general(
        rows_ref[...], w_ref[...], (((1,), (1,)), ((), ())),
        preferred_element_type=jnp.float32)
    t = t + b_ref[...]

    colf_ch = jax.lax.broadcasted_iota(
        jnp.int32, (CH, BR), 0).astype(jnp.float32)
    rowf = (jnp.float32(r0)
            + jax.lax.broadcasted_iota(
                jnp.int32, (CH, BR), 1).astype(jnp.float32))

    def fill(c, _):
        c0 = c * CH
        mc = ments_ref[pl.ds(c0, CH), :]
        s = jax.lax.dot_general(
            mc, t, (((1,), (1,)), ((), ())),
            preferred_element_type=jnp.float32)
        colf = jnp.float32(c0) + colf_ch
        s_ref[pl.ds(c0, CH), :] = jnp.where(colf < rowf, s, NEG)
        return 0

    jax.lax.fori_loop(0, UNROLL * nchu, fill, 0)

    def extract(k, idx_prevf):
        def subchunk(c0, carry):
            m_run, i_run = carry
            chunk = s_ref[pl.ds(c0, CH), :]
            colf = jnp.float32(c0) + colf_ch
            masked = jnp.where(colf == idx_prevf, -jnp.inf, chunk)
            s_ref[pl.ds(c0, CH), :] = masked
            cmax = jnp.max(masked, axis=0, keepdims=True)
            cidx = jnp.min(jnp.where(masked == cmax, colf, jnp.float32(n)),
                           axis=0, keepdims=True)
            upd = cmax > m_run
            return (jnp.where(upd, cmax, m_run),
                    jnp.where(upd, cidx, i_run))

        def scan(c, carry):
            for h in range(UNROLL):
                carry = subchunk((UNROLL * c + h) * CH, carry)
            return carry

        m, idxf = jax.lax.fori_loop(
            0, nchu, scan,
            (jnp.full((1, BR), -jnp.inf, jnp.float32),
             jnp.full((1, BR), jnp.float32(n), jnp.float32)))

        vals_ref[pl.ds(k, 1), :] = jnp.where(m <= NEG * 0.5, -jnp.inf, m)
        idx_ref[pl.ds(k, 1), :] = idxf.astype(jnp.int32)
        return idxf

    jax.lax.fori_loop(0, K, extract, jnp.full((1, BR), -1.0, jnp.float32))
    vals_ref[pl.ds(K, KPAD - K), :] = jnp.zeros((KPAD - K, BR), jnp.float32)
    idx_ref[pl.ds(K, KPAD - K), :] = jnp.zeros((KPAD - K, BR), jnp.int32)


@jax.jit
def kernel(mentions, W, b):
    n, d = mentions.shape
    grid = (n // BR,)
    vals_t, idx_t = pl.pallas_call(
        _rough_topk_kernel,
        grid=grid,
        in_specs=[
            pl.BlockSpec((BR, d), lambda i: (i, 0)),
            pl.BlockSpec((d, d), lambda i: (0, 0)),
            pl.BlockSpec((1, d), lambda i: (0, 0)),
            pl.BlockSpec((n, d), lambda i: (0, 0)),
        ],
        out_specs=[
            pl.BlockSpec((KPAD, BR), lambda i: (0, i)),
            pl.BlockSpec((KPAD, BR), lambda i: (0, i)),
        ],
        out_shape=[
            jax.ShapeDtypeStruct((KPAD, n), jnp.float32),
            jax.ShapeDtypeStruct((KPAD, n), jnp.int32),
        ],
        scratch_shapes=[
            pltpu.VMEM((n, BR), jnp.float32),
        ],
    )(mentions, W, b.reshape(1, d), mentions)
    return vals_t[:K, :].T, idx_t[:K, :].T


# speedup vs baseline: 1.0757x; 1.0001x over previous
import jax
import jax.numpy as jnp
from jax.experimental import pallas as pl
from jax.experimental.pallas import tpu as pltpu

K = 50
BR = 256
CH = 256
UNROLL = 4
KPAD = 64
NEG = -1e38


def _rough_topk_kernel(rows_ref, w_ref, b_ref, ments_ref,
                       vals_ref, idx_ref, s_ref):
    n = ments_ref.shape[0]
    r0 = pl.program_id(0) * BR
    nch = (r0 + BR + CH - 1) // CH
    nchu = (nch + UNROLL - 1) // UNROLL

    t = jax.lax.dot_general(
        rows_ref[...], w_ref[...], (((1,), (1,)), ((), ())),
        preferred_element_type=jnp.float32)
    t = t + b_ref[...]

    colf_ch = jax.lax.broadcasted_iota(
        jnp.int32, (CH, BR), 0).astype(jnp.float32)
    rowf = (jnp.float32(r0)
            + jax.lax.broadcasted_iota(
                jnp.int32, (CH, BR), 1).astype(jnp.float32))

    def fill(c, _):
        c0 = c * CH
        mc = ments_ref[pl.ds(c0, CH), :]
        s = jax.lax.dot_general(
            mc, t, (((1,), (1,)), ((), ())),
            preferred_element_type=jnp.float32)
        colf = jnp.float32(c0) + colf_ch
        s_ref[pl.ds(c0, CH), :] = jnp.where(colf < rowf, s, NEG)
        return 0

    jax.lax.fori_loop(0, UNROLL * nchu, fill, 0)

    def extract(k, idx_prevf):
        def subchunk(c0, carry):
            m_run, i_run = carry
            chunk = s_ref[pl.ds(c0, CH), :]
            colf = jnp.float32(c0) + colf_ch
            masked = jnp.where(colf == idx_prevf, -jnp.inf, chunk)
            s_ref[pl.ds(c0, CH), :] = masked
            cmax = jnp.max(masked, axis=0, keepdims=True)
            cidx = jnp.min(jnp.where(masked == cmax, colf, jnp.float32(n)),
                           axis=0, keepdims=True)
            upd = cmax > m_run
            return (jnp.where(upd, cmax, m_run),
                    jnp.where(upd, cidx, i_run))

        def scan(c, carry):
            for h in range(UNROLL):
                carry = subchunk((UNROLL * c + h) * CH, carry)
            return carry

        m, idxf = jax.lax.fori_loop(
            0, nchu, scan,
            (jnp.full((1, BR), -jnp.inf, jnp.float32),
             jnp.full((1, BR), jnp.float32(n), jnp.float32)))

        vals_ref[pl.ds(k, 1), :] = jnp.where(m <= NEG * 0.5, -jnp.inf, m)
        idx_ref[pl.ds(k, 1), :] = idxf.astype(jnp.int32)
        return idxf

    jax.lax.fori_loop(0, K, extract, jnp.full((1, BR), -1.0, jnp.float32))
    vals_ref[pl.ds(K, KPAD - K), :] = jnp.zeros((KPAD - K, BR), jnp.float32)
    idx_ref[pl.ds(K, KPAD - K), :] = jnp.zeros((KPAD - K, BR), jnp.int32)


@jax.jit
def kernel(mentions, W, b):
    n, d = mentions.shape
    grid = (n // BR,)
    vals_t, idx_t = pl.pallas_call(
        _rough_topk_kernel,
        grid=grid,
        in_specs=[
            pl.BlockSpec((BR, d), lambda i: (i, 0)),
            pl.BlockSpec((d, d), lambda i: (0, 0)),
            pl.BlockSpec((1, d), lambda i: (0, 0)),
            pl.BlockSpec((n, d), lambda i: (0, 0)),
        ],
        out_specs=[
            pl.BlockSpec((KPAD, BR), lambda i: (0, i)),
            pl.BlockSpec((KPAD, BR), lambda i: (0, i)),
        ],
        out_shape=[
            jax.ShapeDtypeStruct((KPAD, n), jnp.float32),
            jax.ShapeDtypeStruct((KPAD, n), jnp.int32),
        ],
        scratch_shapes=[
            pltpu.VMEM((n, BR), jnp.float32),
        ],
    )(mentions, W, b.reshape(1, d), mentions)
    return vals_t[:K, :].T, idx_t[:K, :].T
